# baseline (speedup 1.0000x reference)
import jax
import jax.numpy as jnp
from jax.experimental import pallas as pl
from jax.experimental.pallas import tpu as pltpu

N = 4096
T = 16
BM = 128
NSTEP = N // BM
SPLIT = 8
BK = N // SPLIT
NBUF = 3


def _fused_kernel(xi, xj1, xk, wi, wj1, wj2, wk,
                  gi_h, aj_h, cj_h, gk_h, out,
                  b_gi, b_aj, b_cj, b_gk,
                  yi, y1, y2, yk, sems):
    bf = jnp.bfloat16
    streams = ((gi_h, b_gi), (aj_h, b_aj), (cj_h, b_cj), (gk_h, b_gk))

    def start_copies(step, slot):
        for m, (hbm, buf) in enumerate(streams):
            for s in range(SPLIT):
                pltpu.make_async_copy(
                    hbm.at[pl.ds(step * BM, BM), pl.ds(s * BK, BK)],
                    buf.at[slot, s],
                    sems.at[slot, m, s],
                ).start()

    for w in range(NBUF):
        start_copies(w, w)

    yi[...] = jnp.dot(
        xi[...], wi[...], preferred_element_type=jnp.float32).astype(bf)
    y1[...] = jnp.dot(
        xj1[...], wj1[...], preferred_element_type=jnp.float32).astype(bf)
    y2[...] = jnp.dot(
        xj1[...], wj2[...], preferred_element_type=jnp.float32).astype(bf)
    yk[...] = jnp.dot(
        xk[...], wk[...], preferred_element_type=jnp.float32).astype(bf)
    ys = (yi, y1, y2, yk)

    for step in range(NSTEP):
        slot = step % NBUF
        acc = jnp.zeros((BM, T), dtype=jnp.float32)
        for m, (hbm, buf) in enumerate(streams):
            for s in range(SPLIT):
                pltpu.make_async_copy(
                    hbm.at[pl.ds(step * BM, BM), pl.ds(s * BK, BK)],
                    buf.at[slot, s],
                    sems.at[slot, m, s],
                ).wait()
                acc += jnp.dot(buf[slot, s].astype(bf),
                               ys[m][s * BK:(s + 1) * BK, :],
                               preferred_element_type=jnp.float32)
        out[step * BM:(step + 1) * BM, :] = jnp.maximum(acc, 0.0)
        nxt = step + NBUF
        if nxt < NSTEP:
            start_copies(nxt, slot)


@jax.jit
def kernel(xi, xj1, xj2, xk, Gi2j, Adj2j, coAdj2j, Gk2j, W_i, W_j1, W_j2, W_k):
    del xj2

    vmem_full = pl.BlockSpec(memory_space=pltpu.MemorySpace.VMEM)
    hbm = pl.BlockSpec(memory_space=pl.ANY)
    g_buf = pltpu.VMEM((NBUF, SPLIT, BM, BK), jnp.float32)
    y_scratch = pltpu.VMEM((N, T), jnp.bfloat16)
    out = pl.pallas_call(
        _fused_kernel,
        in_specs=[vmem_full, vmem_full, vmem_full,
                  vmem_full, vmem_full, vmem_full, vmem_full,
                  hbm, hbm, hbm, hbm],
        out_specs=vmem_full,
        out_shape=jax.ShapeDtypeStruct((N, T), jnp.float32),
        scratch_shapes=[g_buf, g_buf, g_buf, g_buf,
                        y_scratch, y_scratch, y_scratch, y_scratch,
                        pltpu.SemaphoreType.DMA((NBUF, 4, SPLIT))],
    )(xi, xj1, xk, W_i, W_j1, W_j2, W_k, Gi2j, Adj2j, coAdj2j, Gk2j)
    return out


# speedup vs baseline: 2.1132x; 2.1132x over previous
import jax
import jax.numpy as jnp
from jax.experimental import pallas as pl
from jax.experimental.pallas import tpu as pltpu

N = 4096
T = 16
BM = 128
SPLIT = 8
BK = N // SPLIT


def _fused_kernel(*refs):
    xi, xj1, xk, wi, wj1, wj2, wk = refs[:7]
    g_refs = refs[7:7 + 4 * SPLIT]
    out = refs[7 + 4 * SPLIT]
    ys = refs[8 + 4 * SPLIT:]
    bf = jnp.bfloat16

    @pl.when(pl.program_id(0) == 0)
    def _compute_projections():
        for y, x, w in ((ys[0], xi, wi), (ys[1], xj1, wj1),
                        (ys[2], xj1, wj2), (ys[3], xk, wk)):
            y[...] = jnp.dot(
                x[...], w[...], preferred_element_type=jnp.float32
            ).astype(bf)

    acc = jnp.zeros((BM, T), dtype=jnp.float32)
    for m in range(4):
        for s in range(SPLIT):
            g = g_refs[m * SPLIT + s]
            y = ys[m][s * BK:(s + 1) * BK, :]
            acc += jnp.dot(g[...].astype(bf), y,
                           preferred_element_type=jnp.float32)
    out[...] = jnp.maximum(acc, 0.0)


@jax.jit
def kernel(xi, xj1, xj2, xk, Gi2j, Adj2j, coAdj2j, Gk2j, W_i, W_j1, W_j2, W_k):
    del xj2

    grid = (N // BM,)
    feat_spec = pl.BlockSpec((N, T), lambda i: (0, 0))
    w_spec = pl.BlockSpec((T, T), lambda i: (0, 0))

    def col_spec(s):
        return pl.BlockSpec((BM, BK), lambda i, s=s: (i, s))

    g_specs = []
    g_args = []
    for G in (Gi2j, Adj2j, coAdj2j, Gk2j):
        for s in range(SPLIT):
            g_specs.append(col_spec(s))
            g_args.append(G)

    y_scratch = pltpu.VMEM((N, T), jnp.bfloat16)
    out = pl.pallas_call(
        _fused_kernel,
        grid=grid,
        in_specs=[feat_spec, feat_spec, feat_spec,
                  w_spec, w_spec, w_spec, w_spec] + g_specs,
        out_specs=pl.BlockSpec((BM, T), lambda i: (i, 0)),
        out_shape=jax.ShapeDtypeStruct((N, T), jnp.float32),
        scratch_shapes=[y_scratch, y_scratch, y_scratch, y_scratch],
        compiler_params=pltpu.CompilerParams(
            dimension_semantics=("arbitrary",),
        ),
    )(xi, xj1, xk, W_i, W_j1, W_j2, W_k, *g_args)
    return out
